# baseline (speedup 1.0000x reference)
import jax
import jax.numpy as jnp
from jax.experimental import pallas as pl

BLOCK_M = 512


def _vq_kernel(x_ref, cb_ref, x2_ref, c2_ref, out_ref):
    x = x_ref[...]
    cb = cb_ref[...]
    x2 = x2_ref[0, 0, :][:, None]
    c2 = c2_ref[0, 0, :]
    m = jax.lax.dot_general(
        x, cb, (((1,), (1,)), ((), ())),
        preferred_element_type=jnp.float32)
    d2 = x2 + c2[None, :] - 2.0 * m
    d = jnp.sqrt(jnp.maximum(d2, 0.0))
    mn = jnp.min(d, axis=1, keepdims=True)
    iota = jax.lax.broadcasted_iota(jnp.int32, d.shape, 1)
    idx = jnp.min(jnp.where(d == mn, iota, jnp.int32(2**30)), axis=1)
    out_ref[0, 0, :] = idx.astype(jnp.int32)


def kernel(z_e_x, codebook):
    b, t, e = z_e_x.shape
    x = z_e_x.reshape(-1, e)
    mrows = x.shape[0]
    n_cb = codebook.shape[0]
    g = mrows // BLOCK_M
    x2 = jnp.sum(x * x, axis=1).reshape(g, 1, BLOCK_M)
    c2 = jnp.sum(codebook * codebook, axis=1).reshape(1, 1, n_cb)
    out = pl.pallas_call(
        _vq_kernel,
        grid=(g,),
        in_specs=[
            pl.BlockSpec((BLOCK_M, e), lambda i: (i, 0)),
            pl.BlockSpec((n_cb, e), lambda i: (0, 0)),
            pl.BlockSpec((1, 1, BLOCK_M), lambda i: (i, 0, 0)),
            pl.BlockSpec((1, 1, n_cb), lambda i: (0, 0, 0)),
        ],
        out_specs=pl.BlockSpec((1, 1, BLOCK_M), lambda i: (i, 0, 0)),
        out_shape=jax.ShapeDtypeStruct((g, 1, BLOCK_M), jnp.int32),
    )(x, codebook, x2, c2)
    return out.reshape(b, t)
